# baseline (speedup 1.0000x reference)
import functools

import jax
import jax.numpy as jnp
from jax import lax
from jax.experimental import pallas as pl
from jax.experimental.pallas import tpu as pltpu
from jax.experimental.pallas import tpu_sc as plsc

_NC = 2
_NS = 16
_NW = _NC * _NS
_L = 16


@functools.lru_cache(maxsize=None)
def _build(S, B, D, V, P):
    rows_w = S // _NW
    nchunk = rows_w // P
    mesh = plsc.VectorSubcoreMesh(
        core_axis_name="c", subcore_axis_name="s",
        num_cores=_NC, num_subcores=_NS,
    )

    @functools.partial(
        pl.kernel,
        out_type=jax.ShapeDtypeStruct((S, B, D), jnp.float32),
        mesh=mesh,
        scratch_types=[
            pltpu.VMEM((nchunk, P), jnp.int32),
            pltpu.VMEM((P, B, D), jnp.float32),
            pltpu.VMEM((P, D), jnp.float32),
            pltpu.SemaphoreType.DMA,
            pltpu.SemaphoreType.DMA,
        ],
    )
    def sc_add(x_hbm, i_hbm, pe_hbm, out_hbm, idx_v, xbuf, pebuf, insem, gsem):
        wid = lax.axis_index("s") * _NC + lax.axis_index("c")
        base = wid * rows_w
        pltpu.sync_copy(i_hbm.at[wid], idx_v)

        def chunk(c, carry):
            pos = base + c * P
            xcp = pltpu.make_async_copy(x_hbm.at[pl.ds(pos, P)], xbuf, insem)
            xcp.start()
            gcp = pltpu.make_async_copy(pe_hbm.at[idx_v.at[c]], pebuf, gsem)
            gcp.start()
            xcp.wait()
            gcp.wait()

            def kbody(k, kcarry):
                sl = pl.ds(k * _L, _L)
                for p in range(P):
                    pv = pebuf[p, sl]
                    for b in range(B):
                        xbuf[p, b, sl] += pv
                return kcarry

            lax.fori_loop(0, D // _L, kbody, 0)
            pltpu.sync_copy(xbuf, out_hbm.at[pl.ds(pos, P)])
            return carry

        lax.fori_loop(0, nchunk, chunk, 0)

    return sc_add


def kernel(x, i, pe):
    S, B, D = x.shape
    V = pe.shape[0]
    P = 8
    i3 = i.astype(jnp.int32).reshape(_NW, (S // _NW) // P, P)
    pe2 = pe.reshape(V, D)
    return _build(S, B, D, V, P)(x, i3, pe2)


# speedup vs baseline: 1.2409x; 1.2409x over previous
import functools

import jax
import jax.numpy as jnp
from jax import lax
from jax.experimental import pallas as pl
from jax.experimental.pallas import tpu as pltpu
from jax.experimental.pallas import tpu_sc as plsc

_NC = 2
_NS = 16
_NW = _NC * _NS
_L = 16


@functools.lru_cache(maxsize=None)
def _build(S, B, D, V, P):
    rows_w = S // _NW
    nchunk = rows_w // P
    mesh = plsc.VectorSubcoreMesh(
        core_axis_name="c", subcore_axis_name="s",
        num_cores=_NC, num_subcores=_NS,
    )

    @functools.partial(
        pl.kernel,
        out_type=jax.ShapeDtypeStruct((S, B, D), jnp.float32),
        mesh=mesh,
        scratch_types=[
            pltpu.VMEM((nchunk, P), jnp.int32),
            [pltpu.VMEM((P, B, D), jnp.float32) for _ in range(2)],
            [pltpu.VMEM((P, D), jnp.float32) for _ in range(2)],
            [pltpu.SemaphoreType.DMA for _ in range(2)],
            [pltpu.SemaphoreType.DMA for _ in range(2)],
            [pltpu.SemaphoreType.DMA for _ in range(2)],
        ],
    )
    def sc_add(x_hbm, i_hbm, pe_hbm, out_hbm, idx_v, xbuf, pebuf, insem, gsem, osem):
        wid = lax.axis_index("s") * _NC + lax.axis_index("c")
        base = wid * rows_w
        pltpu.sync_copy(i_hbm.at[wid], idx_v)

        def in_copies(c):
            t = c % 2
            return (
                pltpu.make_async_copy(
                    x_hbm.at[pl.ds(base + c * P, P)], xbuf[t], insem[t]),
                pltpu.make_async_copy(
                    pe_hbm.at[idx_v.at[c]], pebuf[t], gsem[t]),
            )

        def out_copy(c):
            t = c % 2
            return pltpu.make_async_copy(
                xbuf[t], out_hbm.at[pl.ds(base + c * P, P)], osem[t])

        for cp in in_copies(0):
            cp.start()
        for c in range(nchunk):
            t = c % 2
            if c + 1 < nchunk:
                if c >= 1:
                    out_copy(c - 1).wait()
                for cp in in_copies(c + 1):
                    cp.start()
            for cp in in_copies(c):
                cp.wait()

            xb, pb = xbuf[t], pebuf[t]

            def kbody(k, kcarry, xb=xb, pb=pb):
                sl = pl.ds(k * _L, _L)
                for p in range(P):
                    pv = pb[p, sl]
                    for b in range(B):
                        xb[p, b, sl] += pv
                return kcarry

            lax.fori_loop(0, D // _L, kbody, 0)
            out_copy(c).start()
        out_copy(nchunk - 2).wait()
        out_copy(nchunk - 1).wait()

    return sc_add


def kernel(x, i, pe):
    S, B, D = x.shape
    V = pe.shape[0]
    P = 8
    i3 = i.astype(jnp.int32).reshape(_NW, (S // _NW) // P, P)
    pe2 = pe.reshape(V, D)
    return _build(S, B, D, V, P)(x, i3, pe2)


# speedup vs baseline: 1.6123x; 1.2992x over previous
import functools

import jax
import jax.numpy as jnp
from jax import lax
from jax.experimental import pallas as pl
from jax.experimental.pallas import tpu as pltpu
from jax.experimental.pallas import tpu_sc as plsc

_NC = 2
_NS = 16
_NW = _NC * _NS
_L = 16


@functools.lru_cache(maxsize=None)
def _build(S, B, D, V, P):
    rows_w = S // _NW
    nchunk = rows_w // P
    mesh = plsc.VectorSubcoreMesh(
        core_axis_name="c", subcore_axis_name="s",
        num_cores=_NC, num_subcores=_NS,
    )

    @functools.partial(
        pl.kernel,
        out_type=jax.ShapeDtypeStruct((S, B, D), jnp.float32),
        mesh=mesh,
        scratch_types=[
            pltpu.VMEM((rows_w,), jnp.int32),
            [pltpu.VMEM((P, B, D), jnp.float32) for _ in range(2)],
            [pltpu.VMEM((P, 1, D), jnp.float32) for _ in range(2)],
            [pltpu.SemaphoreType.DMA for _ in range(2)],
            [pltpu.SemaphoreType.DMA for _ in range(2)],
            [pltpu.SemaphoreType.DMA for _ in range(2)],
        ],
    )
    def sc_add(x_hbm, i_hbm, pe_hbm, out_hbm, idx_v, xbuf, pebuf, insem, gsem, osem):
        wid = lax.axis_index("s") * _NC + lax.axis_index("c")
        base = wid * rows_w
        pltpu.sync_copy(i_hbm.at[pl.ds(base, rows_w)], idx_v)

        def in_copies(c):
            t = c % 2
            return (
                pltpu.make_async_copy(
                    x_hbm.at[pl.ds(base + c * P, P)], xbuf[t], insem[t]),
                pltpu.make_async_copy(
                    pe_hbm.at[idx_v.at[pl.ds(c * P, P)]], pebuf[t], gsem[t]),
            )

        def out_copy(c):
            t = c % 2
            return pltpu.make_async_copy(
                xbuf[t], out_hbm.at[pl.ds(base + c * P, P)], osem[t])

        for cp in in_copies(0):
            cp.start()
        for c in range(nchunk):
            t = c % 2
            if c + 1 < nchunk:
                if c >= 1:
                    out_copy(c - 1).wait()
                for cp in in_copies(c + 1):
                    cp.start()
            for cp in in_copies(c):
                cp.wait()

            xb, pb = xbuf[t], pebuf[t]

            def kbody(k, kcarry, xb=xb, pb=pb):
                sl = pl.ds(k * _L, _L)
                for p in range(P):
                    pv = pb[p, 0, sl]
                    for b in range(B):
                        xb[p, b, sl] += pv
                return kcarry

            lax.fori_loop(0, D // _L, kbody, 0)
            out_copy(c).start()
        out_copy(nchunk - 2).wait()
        out_copy(nchunk - 1).wait()

    return sc_add


def kernel(x, i, pe):
    S, B, D = x.shape
    V = pe.shape[0]
    P = 8
    return _build(S, B, D, V, P)(x, i.astype(jnp.int32), pe)
